# baseline (speedup 1.0000x reference)
import functools

import jax
import jax.numpy as jnp
from jax import lax
from jax.experimental import pallas as pl
from jax.experimental.pallas import tpu as pltpu
from jax.experimental.pallas import tpu_sc as plsc

N = 10000
NPAD = 10240
E = 320000
D = 128
NC = 2
NSUB = 16
NW = NC * NSUB
K = 256
R = -(-E // (K * NW))
EPAD = K * NW * R
RPT = NPAD // NSUB

_mesh = plsc.VectorSubcoreMesh(core_axis_name="c", subcore_axis_name="s")


@functools.partial(
    pl.kernel,
    out_type=jax.ShapeDtypeStruct((2 * NPAD, D), jnp.float32),
    mesh=_mesh,
    scratch_types=[
        pltpu.VMEM((K // 64, 64), jnp.int32),
        pltpu.VMEM((K // 64, 64), jnp.int32),
        pltpu.VMEM((K,), jnp.float32),
        pltpu.VMEM((K, D), jnp.float32),
        pltpu.VMEM_SHARED((NPAD, D), jnp.float32),
        pltpu.SemaphoreType.DMA,
    ],
)
def _lhat_sc(u_hbm, row_hbm, col_hbm, ew_hbm, z_hbm, out_hbm,
             row_v, col_v, ew_v, rows_v, acc, sem):
    cid = lax.axis_index("c")
    sid = lax.axis_index("s")
    wid = sid * NC + cid
    part = pl.ds(sid * RPT, RPT)

    pltpu.sync_copy(z_hbm.at[part], acc.at[part])
    plsc.subcore_barrier()

    @pl.loop(0, R)
    def _(r):
        c = wid + NW * r
        d1 = pltpu.async_copy(row_hbm.at[c], row_v, sem)
        d2 = pltpu.async_copy(col_hbm.at[c], col_v, sem)
        d3 = pltpu.async_copy(ew_hbm.at[pl.ds(c * K, K)], ew_v, sem)
        d1.wait()
        d2.wait()
        d3.wait()
        gs = [pltpu.async_copy(u_hbm.at[row_v.at[q]],
                               rows_v.at[pl.ds(q * 64, 64)], sem)
              for q in range(K // 64)]
        for g_ in gs:
            g_.wait()

        @pl.loop(0, K // 16)
        def _(g):
            ewv = ew_v[pl.ds(g * 16, 16)]
            for j in range(16):
                sc = ewv[j]
                for f in range(D // 16):
                    sl = pl.ds(f * 16, 16)
                    rows_v[g * 16 + j, sl] = rows_v[g * 16 + j, sl] * sc

        ss = [pltpu.async_copy(rows_v.at[pl.ds(q * 64, 64)],
                               acc.at[col_v.at[q]], sem, add=True)
              for q in range(K // 64)]
        for s_ in ss:
            s_.wait()

    plsc.subcore_barrier()
    pltpu.sync_copy(acc.at[part],
                    out_hbm.at[pl.ds(cid * NPAD + sid * RPT, RPT)])


@functools.partial(
    pl.kernel,
    out_type=jax.ShapeDtypeStruct((2 * NPAD, D), jnp.float32),
    mesh=_mesh,
    scratch_types=[
        pltpu.VMEM((K // 64, 64), jnp.int32),
        pltpu.VMEM((K,), jnp.float32),
        pltpu.VMEM((K, D), jnp.float32),
        pltpu.VMEM_SHARED((NPAD, D), jnp.float32),
        pltpu.SemaphoreType.DMA,
    ],
)
def _deg_sc(row_hbm, ew_hbm, z_hbm, out_hbm,
            row_v, ew_v, rows_v, acc, sem):
    cid = lax.axis_index("c")
    sid = lax.axis_index("s")
    wid = sid * NC + cid
    part = pl.ds(sid * RPT, RPT)

    pltpu.sync_copy(z_hbm.at[part], acc.at[part])
    pltpu.sync_copy(z_hbm.at[pl.ds(0, K)], rows_v)
    plsc.subcore_barrier()

    @pl.loop(0, R)
    def _(r):
        c = wid + NW * r
        d1 = pltpu.async_copy(row_hbm.at[c], row_v, sem)
        d2 = pltpu.async_copy(ew_hbm.at[pl.ds(c * K, K)], ew_v, sem)
        d1.wait()
        d2.wait()

        @pl.loop(0, K // 16)
        def _(g):
            ewv = ew_v[pl.ds(g * 16, 16)]
            for j in range(16):
                rows_v[g * 16 + j, pl.ds(0, 16)] = jnp.full((16,), ewv[j],
                                                            jnp.float32)

        ss = [pltpu.async_copy(rows_v.at[pl.ds(q * 64, 64)],
                               acc.at[row_v.at[q]], sem, add=True)
              for q in range(K // 64)]
        for s_ in ss:
            s_.wait()

    plsc.subcore_barrier()
    pltpu.sync_copy(acc.at[part],
                    out_hbm.at[pl.ds(cid * NPAD + sid * RPT, RPT)])


BE = 1280
GE = E // BE
BN = 1024
GN = NPAD // BN

_full2 = lambda shape: pl.BlockSpec(shape, lambda i: (0, 0))


def _edgenet_body(xeT_ref, w1t_ref, b1_ref, w2t_ref, b2_ref, out_ref):
    xT = xeT_ref[...]
    w1t = w1t_ref[...]
    b1 = b1_ref[...]
    w2t = w2t_ref[...]
    b2 = b2_ref[...]

    def parser(hT):
        h = jnp.maximum(jnp.dot(w1t, hT, preferred_element_type=jnp.float32) + b1, 0.0)
        return jnp.dot(w2t, h, preferred_element_type=jnp.float32) + b2

    p1 = parser(xT[:16])
    p2 = parser(xT[16:])
    dot = jnp.sum(p1 * p2, axis=0)
    n1 = jnp.sqrt(jnp.sum(p1 * p1, axis=0))
    n2 = jnp.sqrt(jnp.sum(p2 * p2, axis=0))
    cos = dot / (n1 * n2 + 1e-8)
    out_ref[0, 0, :] = (cos + 1.0) * 0.5


_edgenet = pl.pallas_call(
    _edgenet_body,
    grid=(GE,),
    in_specs=[
        pl.BlockSpec((32, BE), lambda i: (0, i)),
        _full2((128, 16)),
        _full2((128, 1)),
        _full2((128, 128)),
        _full2((128, 1)),
    ],
    out_specs=pl.BlockSpec((1, 1, BE), lambda i: (i, 0, 0)),
    out_shape=jax.ShapeDtypeStruct((GE, 1, BE), jnp.float32),
)


def _disu_body(d0_ref, d1_ref, feat_ref, dis_ref, u_ref):
    deg = d0_ref[:, 0] + d1_ref[:, 0] + 1.0
    dis = jnp.where(deg > 0, lax.rsqrt(deg), 0.0)
    disb = jnp.broadcast_to(dis[:, None], (BN, D))
    dis_ref[...] = disb
    u_ref[...] = disb * feat_ref[...]


_disu = pl.pallas_call(
    _disu_body,
    grid=(GN,),
    in_specs=[
        pl.BlockSpec((BN, D), lambda i: (i, 0)),
        pl.BlockSpec((BN, D), lambda i: (i + GN, 0)),
        pl.BlockSpec((BN, D), lambda i: (i, 0)),
    ],
    out_specs=[
        pl.BlockSpec((BN, D), lambda i: (i, 0)),
        pl.BlockSpec((BN, D), lambda i: (i, 0)),
    ],
    out_shape=[
        jax.ShapeDtypeStruct((NPAD, D), jnp.float32),
        jax.ShapeDtypeStruct((NPAD, D), jnp.float32),
    ],
)


def _mid_body(y0_ref, y1_ref, x_ref, dis_ref, w0_ref, w1_ref, u1_ref, acc_ref):
    disb = dis_ref[...]
    x = x_ref[...]
    u_in = disb * x
    tx1 = -disb * (y0_ref[...] + y1_ref[...] + u_in)
    u1_ref[...] = disb * tx1
    acc_ref[...] = (jnp.dot(x, w0_ref[...], preferred_element_type=jnp.float32)
                    + jnp.dot(tx1, w1_ref[...], preferred_element_type=jnp.float32))


_mid = pl.pallas_call(
    _mid_body,
    grid=(GN,),
    in_specs=[pl.BlockSpec((BN, D), lambda i: (i, 0)),
              pl.BlockSpec((BN, D), lambda i: (i + GN, 0)),
              pl.BlockSpec((BN, D), lambda i: (i, 0)),
              pl.BlockSpec((BN, D), lambda i: (i, 0))]
    + [_full2((D, D)), _full2((D, D))],
    out_specs=[
        pl.BlockSpec((BN, D), lambda i: (i, 0)),
        pl.BlockSpec((BN, D), lambda i: (i, 0)),
    ],
    out_shape=[
        jax.ShapeDtypeStruct((NPAD, D), jnp.float32),
        jax.ShapeDtypeStruct((NPAD, D), jnp.float32),
    ],
)


def _end_body(y0_ref, y1_ref, x_ref, dis_ref, acc_ref, xt_ref, u1_ref, w2_ref,
              xt_out_ref, u_out_ref):
    disb = dis_ref[...]
    tx2 = -2.0 * disb * (y0_ref[...] + y1_ref[...] + u1_ref[...]) - x_ref[...]
    xnew = jnp.maximum(
        acc_ref[...] + jnp.dot(tx2, w2_ref[...], preferred_element_type=jnp.float32),
        0.0)
    xt = xt_ref[...] + xnew
    xt_out_ref[...] = xt
    u_out_ref[...] = disb * xt


_end = pl.pallas_call(
    _end_body,
    grid=(GN,),
    in_specs=[pl.BlockSpec((BN, D), lambda i: (i, 0)),
              pl.BlockSpec((BN, D), lambda i: (i + GN, 0))]
    + [pl.BlockSpec((BN, D), lambda i: (i, 0))] * 5 + [_full2((D, D))],
    out_specs=[
        pl.BlockSpec((BN, D), lambda i: (i, 0)),
        pl.BlockSpec((BN, D), lambda i: (i, 0)),
    ],
    out_shape=[
        jax.ShapeDtypeStruct((NPAD, D), jnp.float32),
        jax.ShapeDtypeStruct((NPAD, D), jnp.float32),
    ],
)


def _final_body(y0_ref, y1_ref, x_ref, dis_ref, acc_ref, u1_ref, w2_ref,
                cw1_ref, cb1_ref, bng_ref, bnb_ref, bnm_ref, bnv_ref,
                cw2_ref, cb2_ref, out_ref):
    disb = dis_ref[...]
    tx2 = -2.0 * disb * (y0_ref[...] + y1_ref[...] + u1_ref[...]) - x_ref[...]
    x4 = jnp.maximum(
        acc_ref[...] + jnp.dot(tx2, w2_ref[...], preferred_element_type=jnp.float32),
        0.0)
    h = jnp.maximum(
        jnp.dot(x4, cw1_ref[...], preferred_element_type=jnp.float32) + cb1_ref[...],
        0.0)
    h = (h - bnm_ref[...]) / jnp.sqrt(bnv_ref[...] + 1e-5) * bng_ref[...] + bnb_ref[...]
    out_ref[...] = jnp.dot(h, cw2_ref[...], preferred_element_type=jnp.float32) + cb2_ref[...]


_final = pl.pallas_call(
    _final_body,
    grid=(GN,),
    in_specs=[pl.BlockSpec((BN, D), lambda i: (i, 0)),
              pl.BlockSpec((BN, D), lambda i: (i + GN, 0))]
    + [pl.BlockSpec((BN, D), lambda i: (i, 0))] * 4
    + [_full2((D, D)), _full2((D, 128)), _full2((1, 128)),
       _full2((1, 128)), _full2((1, 128)), _full2((1, 128)), _full2((1, 128)),
       _full2((128, 10)), _full2((1, 10))],
    out_specs=pl.BlockSpec((BN, 10), lambda i: (i, 0)),
    out_shape=jax.ShapeDtypeStruct((NPAD, 10), jnp.float32),
)


def kernel(features, edge_index, edgenet_input, wl_w1, wl_b1, wl_w2, wl_b2,
           cheb_w0, cheb_w1, cheb_w2, cheb_w3,
           cls_w1, cls_b1, bn_g, bn_b, bn_m, bn_v, cls_w2, cls_b2):
    f32 = jnp.float32
    ew_e = _edgenet(edgenet_input.T, wl_w1.T, wl_b1.reshape(-1, 1),
                    wl_w2.T, wl_b2.reshape(-1, 1)).reshape(E)
    edge_weight = jnp.concatenate([ew_e, jnp.ones((N,), f32)])

    pad_i = jnp.zeros((EPAD - E,), edge_index.dtype)
    pad_f = jnp.zeros((EPAD - E,), f32)
    row_p = jnp.concatenate([edge_index[0], pad_i]).reshape(EPAD // K, K // 64, 64)
    col_p = jnp.concatenate([edge_index[1], pad_i]).reshape(EPAD // K, K // 64, 64)
    ew_p = jnp.concatenate([ew_e, pad_f])

    z128 = jnp.zeros((NPAD, D), f32)
    feat_p = jnp.pad(features, ((0, NPAD - N), (0, 0)))

    dp = _deg_sc(row_p, ew_p, z128)
    disb, u = _disu(dp, dp, feat_p)

    Ws = (cheb_w0, cheb_w1, cheb_w2, cheb_w3)
    x_in = feat_p
    xt = z128
    out = None
    for l in range(4):
        w0, w1, w2 = Ws[l][0], Ws[l][1], Ws[l][2]
        ya = _lhat_sc(u, row_p, col_p, ew_p, z128)
        u1, acc = _mid(ya, ya, x_in, disb, w0, w1)
        yb = _lhat_sc(u1, row_p, col_p, ew_p, z128)
        if l < 3:
            xt, u = _end(yb, yb, x_in, disb, acc, xt, u1, w2)
            x_in = xt
        else:
            out = _final(yb, yb, x_in, disb, acc, u1, w2,
                         cls_w1, cls_b1.reshape(1, -1), bn_g.reshape(1, -1),
                         bn_b.reshape(1, -1), bn_m.reshape(1, -1),
                         bn_v.reshape(1, -1), cls_w2, cls_b2.reshape(1, -1))
    return out[:N], edge_weight


# speedup vs baseline: 1.1280x; 1.1280x over previous
import functools

import jax
import jax.numpy as jnp
from jax import lax
from jax.experimental import pallas as pl
from jax.experimental.pallas import tpu as pltpu
from jax.experimental.pallas import tpu_sc as plsc

N = 10000
NPAD = 10240
E = 320000
EN = E + N
D = 128
NC = 2
NSUB = 16
NW = NC * NSUB
K = 256
R = -(-EN // (K * NW))
EPAD = K * NW * R
RPT = NPAD // NSUB

_mesh = plsc.VectorSubcoreMesh(core_axis_name="c", subcore_axis_name="s")


@functools.partial(
    pl.kernel,
    out_type=jax.ShapeDtypeStruct((2 * NPAD, D), jnp.float32),
    mesh=_mesh,
    scratch_types=[
        pltpu.VMEM((K // 128, 128), jnp.int32),
        pltpu.VMEM((K // 128, 128), jnp.int32),
        pltpu.VMEM((K,), jnp.float32),
        pltpu.VMEM((K, D), jnp.float32),
        pltpu.VMEM_SHARED((NPAD, D), jnp.float32),
        pltpu.SemaphoreType.DMA,
    ],
)
def _lhat_sc(u_hbm, row_hbm, col_hbm, ew_hbm, z_hbm, out_hbm,
             row_v, col_v, ew_v, rows_v, acc, sem):
    cid = lax.axis_index("c")
    sid = lax.axis_index("s")
    wid = sid * NC + cid
    part = pl.ds(sid * RPT, RPT)

    pltpu.sync_copy(z_hbm.at[part], acc.at[part])
    plsc.subcore_barrier()

    @pl.loop(0, R)
    def _(r):
        c = wid + NW * r
        d1 = pltpu.async_copy(row_hbm.at[c], row_v, sem)
        d2 = pltpu.async_copy(col_hbm.at[c], col_v, sem)
        d3 = pltpu.async_copy(ew_hbm.at[pl.ds(c * K, K)], ew_v, sem)
        d1.wait()
        d2.wait()
        d3.wait()
        gs = [pltpu.async_copy(u_hbm.at[row_v.at[q]],
                               rows_v.at[pl.ds(q * 128, 128)], sem)
              for q in range(K // 128)]
        for g_ in gs:
            g_.wait()

        @pl.loop(0, K // 16)
        def _(g):
            ewv = ew_v[pl.ds(g * 16, 16)]
            for j in range(16):
                s = ewv[j]
                for f in range(D // 16):
                    sl = pl.ds(f * 16, 16)
                    rows_v[g * 16 + j, sl] = rows_v[g * 16 + j, sl] * s

        ss = [pltpu.async_copy(rows_v.at[pl.ds(q * 128, 128)],
                               acc.at[col_v.at[q]], sem, add=True)
              for q in range(K // 128)]
        for s_ in ss:
            s_.wait()

    plsc.subcore_barrier()
    pltpu.sync_copy(acc.at[part],
                    out_hbm.at[pl.ds(cid * NPAD + sid * RPT, RPT)])


@functools.partial(
    pl.kernel,
    out_type=jax.ShapeDtypeStruct((2 * NPAD, D), jnp.float32),
    mesh=_mesh,
    scratch_types=[
        pltpu.VMEM((K // 128, 128), jnp.int32),
        pltpu.VMEM((K,), jnp.float32),
        pltpu.VMEM((K, D), jnp.float32),
        pltpu.VMEM_SHARED((NPAD, D), jnp.float32),
        pltpu.SemaphoreType.DMA,
    ],
)
def _deg_sc(row_hbm, ew_hbm, z_hbm, out_hbm,
            row_v, ew_v, rows_v, acc, sem):
    cid = lax.axis_index("c")
    sid = lax.axis_index("s")
    wid = sid * NC + cid
    part = pl.ds(sid * RPT, RPT)

    pltpu.sync_copy(z_hbm.at[part], acc.at[part])
    pltpu.sync_copy(z_hbm.at[pl.ds(0, K)], rows_v)
    plsc.subcore_barrier()

    @pl.loop(0, R)
    def _(r):
        c = wid + NW * r
        d1 = pltpu.async_copy(row_hbm.at[c], row_v, sem)
        d2 = pltpu.async_copy(ew_hbm.at[pl.ds(c * K, K)], ew_v, sem)
        d1.wait()
        d2.wait()

        @pl.loop(0, K // 16)
        def _(g):
            ewv = ew_v[pl.ds(g * 16, 16)]
            for j in range(16):
                rows_v[g * 16 + j, pl.ds(0, 16)] = jnp.full((16,), ewv[j],
                                                            jnp.float32)

        ss = [pltpu.async_copy(rows_v.at[pl.ds(q * 128, 128)],
                               acc.at[row_v.at[q]], sem, add=True)
              for q in range(K // 128)]
        for s_ in ss:
            s_.wait()

    plsc.subcore_barrier()
    pltpu.sync_copy(acc.at[part],
                    out_hbm.at[pl.ds(cid * NPAD + sid * RPT, RPT)])


BE = 1280
GE = E // BE
BN = 1024
GN = NPAD // BN

_full2 = lambda shape: pl.BlockSpec(shape, lambda i: (0, 0))


def _edgenet_body(xeT_ref, w1t_ref, b1_ref, w2t_ref, b2_ref, out_ref):
    xT = xeT_ref[...]
    w1t = w1t_ref[...]
    b1 = b1_ref[...]
    w2t = w2t_ref[...]
    b2 = b2_ref[...]

    def parser(hT):
        h = jnp.maximum(jnp.dot(w1t, hT, preferred_element_type=jnp.float32) + b1, 0.0)
        return jnp.dot(w2t, h, preferred_element_type=jnp.float32) + b2

    p1 = parser(xT[:16])
    p2 = parser(xT[16:])
    dot = jnp.sum(p1 * p2, axis=0)
    n1 = jnp.sqrt(jnp.sum(p1 * p1, axis=0))
    n2 = jnp.sqrt(jnp.sum(p2 * p2, axis=0))
    cos = dot / (n1 * n2 + 1e-8)
    out_ref[0, 0, :] = (cos + 1.0) * 0.5


_edgenet = pl.pallas_call(
    _edgenet_body,
    grid=(GE,),
    in_specs=[
        pl.BlockSpec((32, BE), lambda i: (0, i)),
        _full2((128, 16)),
        _full2((128, 1)),
        _full2((128, 128)),
        _full2((128, 1)),
    ],
    out_specs=pl.BlockSpec((1, 1, BE), lambda i: (i, 0, 0)),
    out_shape=jax.ShapeDtypeStruct((GE, 1, BE), jnp.float32),
)


def _disu_body(d0_ref, d1_ref, feat_ref, dis_ref, u_ref):
    deg = d0_ref[:, 0] + d1_ref[:, 0]
    dis = jnp.where(deg > 0, lax.rsqrt(deg), 0.0)
    disb = jnp.broadcast_to(dis[:, None], (BN, D))
    dis_ref[...] = disb
    u_ref[...] = disb * feat_ref[...]


_disu = pl.pallas_call(
    _disu_body,
    grid=(GN,),
    in_specs=[
        pl.BlockSpec((BN, D), lambda i: (i, 0)),
        pl.BlockSpec((BN, D), lambda i: (i + GN, 0)),
        pl.BlockSpec((BN, D), lambda i: (i, 0)),
    ],
    out_specs=[
        pl.BlockSpec((BN, D), lambda i: (i, 0)),
        pl.BlockSpec((BN, D), lambda i: (i, 0)),
    ],
    out_shape=[
        jax.ShapeDtypeStruct((NPAD, D), jnp.float32),
        jax.ShapeDtypeStruct((NPAD, D), jnp.float32),
    ],
)


def _mid_body(y0_ref, y1_ref, x_ref, dis_ref, w0_ref, w1_ref, u1_ref, acc_ref):
    disb = dis_ref[...]
    tx1 = -disb * (y0_ref[...] + y1_ref[...])
    u1_ref[...] = disb * tx1
    acc_ref[...] = (jnp.dot(x_ref[...], w0_ref[...], preferred_element_type=jnp.float32)
                    + jnp.dot(tx1, w1_ref[...], preferred_element_type=jnp.float32))


_mid = pl.pallas_call(
    _mid_body,
    grid=(GN,),
    in_specs=[pl.BlockSpec((BN, D), lambda i: (i, 0)),
              pl.BlockSpec((BN, D), lambda i: (i + GN, 0)),
              pl.BlockSpec((BN, D), lambda i: (i, 0)),
              pl.BlockSpec((BN, D), lambda i: (i, 0))]
    + [_full2((D, D)), _full2((D, D))],
    out_specs=[
        pl.BlockSpec((BN, D), lambda i: (i, 0)),
        pl.BlockSpec((BN, D), lambda i: (i, 0)),
    ],
    out_shape=[
        jax.ShapeDtypeStruct((NPAD, D), jnp.float32),
        jax.ShapeDtypeStruct((NPAD, D), jnp.float32),
    ],
)


def _end_body(y0_ref, y1_ref, x_ref, dis_ref, acc_ref, xt_ref, w2_ref,
              xt_out_ref, u_out_ref):
    disb = dis_ref[...]
    tx2 = -2.0 * disb * (y0_ref[...] + y1_ref[...]) - x_ref[...]
    xnew = jnp.maximum(
        acc_ref[...] + jnp.dot(tx2, w2_ref[...], preferred_element_type=jnp.float32),
        0.0)
    xt = xt_ref[...] + xnew
    xt_out_ref[...] = xt
    u_out_ref[...] = disb * xt


_end = pl.pallas_call(
    _end_body,
    grid=(GN,),
    in_specs=[pl.BlockSpec((BN, D), lambda i: (i, 0)),
              pl.BlockSpec((BN, D), lambda i: (i + GN, 0))]
    + [pl.BlockSpec((BN, D), lambda i: (i, 0))] * 4 + [_full2((D, D))],
    out_specs=[
        pl.BlockSpec((BN, D), lambda i: (i, 0)),
        pl.BlockSpec((BN, D), lambda i: (i, 0)),
    ],
    out_shape=[
        jax.ShapeDtypeStruct((NPAD, D), jnp.float32),
        jax.ShapeDtypeStruct((NPAD, D), jnp.float32),
    ],
)


def _final_body(y0_ref, y1_ref, x_ref, dis_ref, acc_ref, w2_ref,
                cw1_ref, cb1_ref, bng_ref, bnb_ref, bnm_ref, bnv_ref,
                cw2_ref, cb2_ref, out_ref):
    disb = dis_ref[...]
    tx2 = -2.0 * disb * (y0_ref[...] + y1_ref[...]) - x_ref[...]
    x4 = jnp.maximum(
        acc_ref[...] + jnp.dot(tx2, w2_ref[...], preferred_element_type=jnp.float32),
        0.0)
    h = jnp.maximum(
        jnp.dot(x4, cw1_ref[...], preferred_element_type=jnp.float32) + cb1_ref[...],
        0.0)
    h = (h - bnm_ref[...]) / jnp.sqrt(bnv_ref[...] + 1e-5) * bng_ref[...] + bnb_ref[...]
    out_ref[...] = jnp.dot(h, cw2_ref[...], preferred_element_type=jnp.float32) + cb2_ref[...]


_final = pl.pallas_call(
    _final_body,
    grid=(GN,),
    in_specs=[pl.BlockSpec((BN, D), lambda i: (i, 0)),
              pl.BlockSpec((BN, D), lambda i: (i + GN, 0))]
    + [pl.BlockSpec((BN, D), lambda i: (i, 0))] * 3
    + [_full2((D, D)), _full2((D, 128)), _full2((1, 128)),
       _full2((1, 128)), _full2((1, 128)), _full2((1, 128)), _full2((1, 128)),
       _full2((128, 10)), _full2((1, 10))],
    out_specs=pl.BlockSpec((BN, 10), lambda i: (i, 0)),
    out_shape=jax.ShapeDtypeStruct((NPAD, 10), jnp.float32),
)


def kernel(features, edge_index, edgenet_input, wl_w1, wl_b1, wl_w2, wl_b2,
           cheb_w0, cheb_w1, cheb_w2, cheb_w3,
           cls_w1, cls_b1, bn_g, bn_b, bn_m, bn_v, cls_w2, cls_b2):
    f32 = jnp.float32
    ew_e = _edgenet(edgenet_input.T, wl_w1.T, wl_b1.reshape(-1, 1),
                    wl_w2.T, wl_b2.reshape(-1, 1)).reshape(E)
    edge_weight = jnp.concatenate([ew_e, jnp.ones((N,), f32)])

    loop = jnp.arange(N, dtype=edge_index.dtype)
    pad_i = jnp.zeros((EPAD - EN,), edge_index.dtype)
    pad_f = jnp.zeros((EPAD - EN,), f32)
    row_p = jnp.concatenate([edge_index[0], loop, pad_i]).reshape(EPAD // K, K // 128, 128)
    col_p = jnp.concatenate([edge_index[1], loop, pad_i]).reshape(EPAD // K, K // 128, 128)
    ew_p = jnp.concatenate([edge_weight, pad_f])

    z128 = jnp.zeros((NPAD, D), f32)
    feat_p = jnp.pad(features, ((0, NPAD - N), (0, 0)))

    dp = _deg_sc(row_p, ew_p, z128)
    disb, u = _disu(dp, dp, feat_p)

    Ws = (cheb_w0, cheb_w1, cheb_w2, cheb_w3)
    x_in = feat_p
    xt = z128
    out = None
    for l in range(4):
        w0, w1, w2 = Ws[l][0], Ws[l][1], Ws[l][2]
        ya = _lhat_sc(u, row_p, col_p, ew_p, z128)
        u1, acc = _mid(ya, ya, x_in, disb, w0, w1)
        yb = _lhat_sc(u1, row_p, col_p, ew_p, z128)
        if l < 3:
            xt, u = _end(yb, yb, x_in, disb, acc, xt, w2)
            x_in = xt
        else:
            out = _final(yb, yb, x_in, disb, acc, w2,
                         cls_w1, cls_b1.reshape(1, -1), bn_g.reshape(1, -1),
                         bn_b.reshape(1, -1), bn_m.reshape(1, -1),
                         bn_v.reshape(1, -1), cls_w2, cls_b2.reshape(1, -1))
    return out[:N], edge_weight
